# Initial kernel scaffold; baseline (speedup 1.0000x reference)
import jax
import jax.numpy as jnp
from jax.experimental import pallas as pl


def kernel(prev_embed, A, S, W_h_w, W_h_b, W_struct_w, W_struct_b, W_rec_w, W_rec_b, W_t_w, W_t_b, sim, time_delta_uv, node1, node2):
    raise NotImplementedError("write your pallas kernel here")


# speedup vs baseline: 1.2007x; 1.2007x over previous
import jax
import jax.numpy as jnp
from jax.experimental import pallas as pl
from jax.experimental.pallas import tpu as pltpu

N = 2048
H = 128
KB = 256
NB = N // KB
GAMMA = 0.5


def _hop_kernel(uv_ref, a_ref, ar1_ref, ar2_ref, sr1_ref, sr2_ref,
                arows_ref, srows_ref, m2_ref):
    i = pl.program_id(0)
    w = jnp.concatenate([ar1_ref[0], ar2_ref[0]], axis=0)
    arows_ref[...] = w
    srows_ref[...] = jnp.concatenate([sr1_ref[0], sr2_ref[0]], axis=0)
    part = jnp.dot(w, a_ref[...], preferred_element_type=jnp.float32)

    @pl.when(i == 0)
    def _():
        m2_ref[...] = part

    @pl.when(i != 0)
    def _():
        m2_ref[...] = m2_ref[...] + part


def _final_kernel(uv_ref, pe_ref, arows_ref, srows_ref, m2_ref,
                  whT_ref, whb_ref, wsT_ref, wsb_ref, wrT_ref, wrb_ref,
                  wtT_ref, wtb_ref, sim_ref, td_ref, out_ref):
    pe = pe_ref[...]
    h_prev = jnp.dot(pe, whT_ref[...],
                     preferred_element_type=jnp.float32) + whb_ref[...]
    six = jnp.concatenate([arows_ref[...], m2_ref[...], srows_ref[...]],
                          axis=0)
    sixT = six.T
    a_col = sixT[:, 0:2]
    m2_col = sixT[:, 2:4]
    s_col = sixT[:, 4:6]
    mask = jnp.logical_or(a_col > 0, m2_col > 0)
    base = (1.0 - GAMMA) * sim_ref[0, 0]
    q = jnp.where(mask, jnp.exp(base + GAMMA * s_col), 0.0)
    qs = jnp.sum(q, axis=0, keepdims=True) + 1e-7
    qn = q / qs
    nn = jnp.sum(mask.astype(jnp.float32), axis=0, keepdims=True)
    hs = []
    for c in (0, 1):
        cc = 1 - c
        x = qn[:, cc:cc + 1] * h_prev
        x = jnp.where(mask[:, cc:cc + 1], x, -1e30)
        m = jnp.max(x, axis=0, keepdims=True)
        h = jax.nn.sigmoid(m)
        h = jnp.where(nn[0, cc] > 0, h, jnp.zeros_like(h))
        hs.append(h)
    h_struct = jnp.concatenate(hs, axis=0)
    h1 = jnp.dot(h_struct, wsT_ref[...],
                 preferred_element_type=jnp.float32) + wsb_ref[...]
    u1 = uv_ref[0]
    u2 = uv_ref[1]
    pe_rows = jnp.concatenate(
        [pe_ref[pl.ds(u1, 1), :], pe_ref[pl.ds(u2, 1), :]], axis=0)
    h2 = jnp.dot(pe_rows, wrT_ref[...],
                 preferred_element_type=jnp.float32) + wrb_ref[...]
    h3 = jnp.dot(td_ref[...], wtT_ref[...],
                 preferred_element_type=jnp.float32) + wtb_ref[...]
    z = jax.nn.sigmoid(h1 + h2 + h3)
    out_ref[...] = pe
    out_ref[pl.ds(u1, 1), :] = z[0:1, :]
    out_ref[pl.ds(u2, 1), :] = z[1:2, :]


def _full_spec(shape):
    return pl.BlockSpec(shape, lambda i, uv, n=len(shape): (0,) * n)


def kernel(prev_embed, A, S, W_h_w, W_h_b, W_struct_w, W_struct_b,
           W_rec_w, W_rec_b, W_t_w, W_t_b, sim, time_delta_uv, node1, node2):
    uv = jnp.stack([jnp.asarray(node1, jnp.int32),
                    jnp.asarray(node2, jnp.int32)])
    A3 = A.reshape(N, 1, N)
    S3 = S.reshape(N, 1, N)

    grid1 = pltpu.PrefetchScalarGridSpec(
        num_scalar_prefetch=1,
        grid=(NB,),
        in_specs=[
            pl.BlockSpec((KB, N), lambda i, uv: (i, 0)),
            pl.BlockSpec((1, 1, KB), lambda i, uv: (uv[0], 0, i)),
            pl.BlockSpec((1, 1, KB), lambda i, uv: (uv[1], 0, i)),
            pl.BlockSpec((1, 1, KB), lambda i, uv: (uv[0], 0, i)),
            pl.BlockSpec((1, 1, KB), lambda i, uv: (uv[1], 0, i)),
        ],
        out_specs=[
            pl.BlockSpec((2, KB), lambda i, uv: (0, i)),
            pl.BlockSpec((2, KB), lambda i, uv: (0, i)),
            pl.BlockSpec((2, N), lambda i, uv: (0, 0)),
        ],
    )
    arows, srows, m2 = pl.pallas_call(
        _hop_kernel,
        grid_spec=grid1,
        out_shape=[
            jax.ShapeDtypeStruct((2, N), jnp.float32),
            jax.ShapeDtypeStruct((2, N), jnp.float32),
            jax.ShapeDtypeStruct((2, N), jnp.float32),
        ],
    )(uv, A, A3, A3, S3, S3)

    whT = W_h_w.T
    wsT = W_struct_w.T
    wrT = W_rec_w.T
    wtT = W_t_w.T
    whb = W_h_b.reshape(1, H)
    wsb = W_struct_b.reshape(1, H)
    wrb = W_rec_b.reshape(1, H)
    wtb = W_t_b.reshape(1, H)
    sim1 = jnp.reshape(sim, (1, 1)).astype(jnp.float32)

    grid2 = pltpu.PrefetchScalarGridSpec(
        num_scalar_prefetch=1,
        grid=(1,),
        in_specs=[
            _full_spec((N, H)),
            _full_spec((2, N)),
            _full_spec((2, N)),
            _full_spec((2, N)),
            _full_spec((H, H)),
            _full_spec((1, H)),
            _full_spec((H, H)),
            _full_spec((1, H)),
            _full_spec((H, H)),
            _full_spec((1, H)),
            _full_spec((4, H)),
            _full_spec((1, H)),
            _full_spec((1, 1)),
            _full_spec((2, 4)),
        ],
        out_specs=[_full_spec((N, H))],
    )
    (z_new,) = pl.pallas_call(
        _final_kernel,
        grid_spec=grid2,
        out_shape=[jax.ShapeDtypeStruct((N, H), jnp.float32)],
    )(uv, prev_embed, arows, srows, m2,
      whT, whb, wsT, wsb, wrT, wrb, wtT, wtb, sim1, time_delta_uv)
    return z_new


# speedup vs baseline: 1.3296x; 1.1073x over previous
import jax
import jax.numpy as jnp
from jax import lax
from jax.experimental import pallas as pl
from jax.experimental.pallas import tpu as pltpu

N = 2048
H = 128
KB = 512
NB = N // KB
GAMMA = 0.5


def _rt(x, w):
    return lax.dot_general(x, w, (((1,), (1,)), ((), ())),
                           preferred_element_type=jnp.float32)


def _kernel(uv_ref, a_ref, ar1_ref, ar2_ref, sr1_ref, sr2_ref, pe_ref,
            wh_ref, whb_ref, ws_ref, wsb_ref, wr_ref, wrb_ref,
            wt_ref, wtb_ref, sim_ref, td_ref,
            out_ref, arows_ref, srows_ref, m2_ref):
    i = pl.program_id(0)

    @pl.when(i < NB)
    def _stream():
        w = jnp.concatenate([ar1_ref[0], ar2_ref[0]], axis=0)
        arows_ref[pl.ds(0, 2), pl.ds(i * KB, KB)] = w
        srows_ref[pl.ds(0, 2), pl.ds(i * KB, KB)] = jnp.concatenate(
            [sr1_ref[0], sr2_ref[0]], axis=0)
        part = jnp.dot(w, a_ref[...], preferred_element_type=jnp.float32)

        @pl.when(i == 0)
        def _():
            m2_ref[...] = part

        @pl.when(i != 0)
        def _():
            m2_ref[...] = m2_ref[...] + part

    @pl.when(i == NB)
    def _finalize():
        pe = pe_ref[...]
        h_prev = _rt(pe, wh_ref[...]) + whb_ref[...]
        six = jnp.concatenate(
            [arows_ref[...], m2_ref[...], srows_ref[...]], axis=0)
        sixT = six.T
        a_col = sixT[:, 0:2]
        m2_col = sixT[:, 2:4]
        s_col = sixT[:, 4:6]
        mask = jnp.logical_or(a_col > 0, m2_col > 0)
        base = (1.0 - GAMMA) * sim_ref[0, 0]
        q = jnp.where(mask, jnp.exp(base + GAMMA * s_col), 0.0)
        qs = jnp.sum(q, axis=0, keepdims=True) + 1e-7
        qn = q / qs
        nn = jnp.sum(mask.astype(jnp.float32), axis=0, keepdims=True)
        hs = []
        for c in (0, 1):
            cc = 1 - c
            x = qn[:, cc:cc + 1] * h_prev
            x = jnp.where(mask[:, cc:cc + 1], x, -1e30)
            m = jnp.max(x, axis=0, keepdims=True)
            h = jax.nn.sigmoid(m)
            h = jnp.where(nn[0, cc] > 0, h, jnp.zeros_like(h))
            hs.append(h)
        h_struct = jnp.concatenate(hs, axis=0)
        h1 = _rt(h_struct, ws_ref[...]) + wsb_ref[...]
        u1 = uv_ref[0]
        u2 = uv_ref[1]
        pe_rows = jnp.concatenate(
            [pe_ref[pl.ds(u1, 1), :], pe_ref[pl.ds(u2, 1), :]], axis=0)
        h2 = _rt(pe_rows, wr_ref[...]) + wrb_ref[...]
        h3 = _rt(td_ref[...], wt_ref[...]) + wtb_ref[...]
        z = jax.nn.sigmoid(h1 + h2 + h3)
        out_ref[...] = pe
        out_ref[pl.ds(u1, 1), :] = z[0:1, :]
        out_ref[pl.ds(u2, 1), :] = z[1:2, :]


def _pin(shape):
    return pl.BlockSpec(shape, lambda i, uv, n=len(shape): (0,) * n)


def kernel(prev_embed, A, S, W_h_w, W_h_b, W_struct_w, W_struct_b,
           W_rec_w, W_rec_b, W_t_w, W_t_b, sim, time_delta_uv, node1, node2):
    uv = jnp.stack([jnp.asarray(node1, jnp.int32),
                    jnp.asarray(node2, jnp.int32)])
    A3 = A.reshape(N, 1, N)
    S3 = S.reshape(N, 1, N)
    whb = W_h_b.reshape(1, H)
    wsb = W_struct_b.reshape(1, H)
    wrb = W_rec_b.reshape(1, H)
    wtb = W_t_b.reshape(1, H)
    sim1 = jnp.reshape(sim, (1, 1)).astype(jnp.float32)

    last = NB - 1
    grid = pltpu.PrefetchScalarGridSpec(
        num_scalar_prefetch=1,
        grid=(NB + 1,),
        in_specs=[
            pl.BlockSpec((KB, N), lambda i, uv: (jnp.minimum(i, last), 0)),
            pl.BlockSpec((1, 1, KB),
                         lambda i, uv: (uv[0], 0, jnp.minimum(i, last))),
            pl.BlockSpec((1, 1, KB),
                         lambda i, uv: (uv[1], 0, jnp.minimum(i, last))),
            pl.BlockSpec((1, 1, KB),
                         lambda i, uv: (uv[0], 0, jnp.minimum(i, last))),
            pl.BlockSpec((1, 1, KB),
                         lambda i, uv: (uv[1], 0, jnp.minimum(i, last))),
            _pin((N, H)),
            _pin((H, H)),
            _pin((1, H)),
            _pin((H, H)),
            _pin((1, H)),
            _pin((H, H)),
            _pin((1, H)),
            _pin((H, 4)),
            _pin((1, H)),
            _pin((1, 1)),
            _pin((2, 4)),
        ],
        out_specs=[_pin((N, H))],
        scratch_shapes=[
            pltpu.VMEM((2, N), jnp.float32),
            pltpu.VMEM((2, N), jnp.float32),
            pltpu.VMEM((2, N), jnp.float32),
        ],
    )
    (z_new,) = pl.pallas_call(
        _kernel,
        grid_spec=grid,
        out_shape=[jax.ShapeDtypeStruct((N, H), jnp.float32)],
    )(uv, A, A3, A3, S3, S3, prev_embed,
      W_h_w, whb, W_struct_w, wsb, W_rec_w, wrb, W_t_w, wtb,
      sim1, time_delta_uv)
    return z_new


# speedup vs baseline: 3.9125x; 2.9427x over previous
import jax
import jax.numpy as jnp
from jax import lax
from jax.experimental import pallas as pl
from jax.experimental.pallas import tpu as pltpu

N = 2048
H = 128
KB = 1024
NB = N // KB
GAMMA = 0.5


def _rt(x, w):
    return lax.dot_general(x, w, (((1,), (1,)), ((), ())),
                           preferred_element_type=jnp.float32)


def _kernel(uv_ref, a_ref, ar1_ref, ar2_ref, sr1_ref, sr2_ref, pe_ref,
            wh_ref, whb_ref, ws_ref, wsb_ref, wr_ref, wrb_ref,
            wt_ref, wtb_ref, sim_ref, td_ref,
            out_ref, arows_ref, srows_ref, m2_ref):
    i = pl.program_id(0)

    r1 = uv_ref[0] % 8
    r2 = uv_ref[1] % 8

    @pl.when(i < NB)
    def _stream():
        w = jnp.concatenate(
            [ar1_ref[pl.ds(r1, 1), :], ar2_ref[pl.ds(r2, 1), :]], axis=0)
        arows_ref[pl.ds(0, 2), pl.ds(i * KB, KB)] = w
        srows_ref[pl.ds(0, 2), pl.ds(i * KB, KB)] = jnp.concatenate(
            [sr1_ref[pl.ds(r1, 1), :], sr2_ref[pl.ds(r2, 1), :]], axis=0)
        part = jnp.dot(w, a_ref[...], preferred_element_type=jnp.float32)

        @pl.when(i == 0)
        def _():
            m2_ref[...] = part

        @pl.when(i != 0)
        def _():
            m2_ref[...] = m2_ref[...] + part

    @pl.when(i == NB)
    def _finalize():
        pe = pe_ref[...]
        h_prev = _rt(pe, wh_ref[...]) + whb_ref[...]
        six = jnp.concatenate(
            [arows_ref[...], m2_ref[...], srows_ref[...]], axis=0)
        sixT = six.T
        a_col = sixT[:, 0:2]
        m2_col = sixT[:, 2:4]
        s_col = sixT[:, 4:6]
        mask = jnp.logical_or(a_col > 0, m2_col > 0)
        base = (1.0 - GAMMA) * sim_ref[0, 0]
        q = jnp.where(mask, jnp.exp(base + GAMMA * s_col), 0.0)
        qs = jnp.sum(q, axis=0, keepdims=True) + 1e-7
        qn = q / qs
        nn = jnp.sum(mask.astype(jnp.float32), axis=0, keepdims=True)
        hs = []
        for c in (0, 1):
            cc = 1 - c
            x = qn[:, cc:cc + 1] * h_prev
            x = jnp.where(mask[:, cc:cc + 1], x, -1e30)
            m = jnp.max(x, axis=0, keepdims=True)
            h = jax.nn.sigmoid(m)
            h = jnp.where(nn[0, cc] > 0, h, jnp.zeros_like(h))
            hs.append(h)
        h_struct = jnp.concatenate(hs, axis=0)
        h1 = _rt(h_struct, ws_ref[...]) + wsb_ref[...]
        u1 = uv_ref[0]
        u2 = uv_ref[1]
        pe_rows = jnp.concatenate(
            [pe_ref[pl.ds(u1, 1), :], pe_ref[pl.ds(u2, 1), :]], axis=0)
        h2 = _rt(pe_rows, wr_ref[...]) + wrb_ref[...]
        h3 = _rt(td_ref[...], wt_ref[...]) + wtb_ref[...]
        z = jax.nn.sigmoid(h1 + h2 + h3)
        out_ref[...] = pe
        out_ref[pl.ds(u1, 1), :] = z[0:1, :]
        out_ref[pl.ds(u2, 1), :] = z[1:2, :]


def _pin(shape):
    return pl.BlockSpec(shape, lambda i, uv, n=len(shape): (0,) * n)


def kernel(prev_embed, A, S, W_h_w, W_h_b, W_struct_w, W_struct_b,
           W_rec_w, W_rec_b, W_t_w, W_t_b, sim, time_delta_uv, node1, node2):
    uv = jnp.stack([jnp.asarray(node1, jnp.int32),
                    jnp.asarray(node2, jnp.int32)])
    whb = W_h_b.reshape(1, H)
    wsb = W_struct_b.reshape(1, H)
    wrb = W_rec_b.reshape(1, H)
    wtb = W_t_b.reshape(1, H)
    sim1 = jnp.reshape(sim, (1, 1)).astype(jnp.float32)

    last = NB - 1
    grid = pltpu.PrefetchScalarGridSpec(
        num_scalar_prefetch=1,
        grid=(NB + 1,),
        in_specs=[
            pl.BlockSpec((KB, N), lambda i, uv: (jnp.minimum(i, last), 0)),
            pl.BlockSpec((8, KB),
                         lambda i, uv: (uv[0] // 8, jnp.minimum(i, last))),
            pl.BlockSpec((8, KB),
                         lambda i, uv: (uv[1] // 8, jnp.minimum(i, last))),
            pl.BlockSpec((8, KB),
                         lambda i, uv: (uv[0] // 8, jnp.minimum(i, last))),
            pl.BlockSpec((8, KB),
                         lambda i, uv: (uv[1] // 8, jnp.minimum(i, last))),
            _pin((N, H)),
            _pin((H, H)),
            _pin((1, H)),
            _pin((H, H)),
            _pin((1, H)),
            _pin((H, H)),
            _pin((1, H)),
            _pin((H, 4)),
            _pin((1, H)),
            _pin((1, 1)),
            _pin((2, 4)),
        ],
        out_specs=[_pin((N, H))],
        scratch_shapes=[
            pltpu.VMEM((2, N), jnp.float32),
            pltpu.VMEM((2, N), jnp.float32),
            pltpu.VMEM((2, N), jnp.float32),
        ],
    )
    (z_new,) = pl.pallas_call(
        _kernel,
        grid_spec=grid,
        out_shape=[jax.ShapeDtypeStruct((N, H), jnp.float32)],
    )(uv, A, A, A, S, S, prev_embed,
      W_h_w, whb, W_struct_w, wsb, W_rec_w, wrb, W_t_w, wtb,
      sim1, time_delta_uv)
    return z_new
